# baseline (speedup 1.0000x reference)
import jax
import jax.numpy as jnp
from jax import lax
from jax.experimental import pallas as pl
from jax.experimental.pallas import tpu as pltpu
from jax.experimental.pallas import tpu_sc as plsc

S = 512
B = 16
F = 30
T = 64
FEAT = 1000000

NW = 32
CHUNKS = S // NW
CW = B * F
RPW = 5
CROWS = CW * RPW


def _sc_body(
    words_hbm, w16_hbm, feats_hbm,
    words_v, r0_v, r1_v, idx0_v, idx1_v, buf0_v, buf1_v, out_v, out2_v,
    sem0, sem1,
):
    wid = lax.axis_index("s") * 2 + lax.axis_index("c")
    iota = lax.iota(jnp.int32, 16)
    bufs = ((r0_v, idx0_v, buf0_v, sem0), (r1_v, idx1_v, buf1_v, sem1))

    def prep(c, par):
        r_v, idx_v, buf_v, sem = bufs[par]
        g = wid * CHUNKS + c
        pltpu.sync_copy(words_hbm.at[pl.ds(g * CW, CW)], words_v)

        def build_f(f, carry):
            wv = plsc.load_gather(words_v, [iota * F + f])
            q = lax.shift_right_logical(wv, 4)
            r_v[pl.ds(f * 16, 16)] = jnp.bitwise_and(wv, 15)
            fvec = jnp.full((16,), f, jnp.int32)
            for k in range(RPW):
                plsc.store_scatter(idx_v, [fvec, iota * RPW + k], q + k)
            return carry

        lax.fori_loop(0, F, build_f, 0)
        for f in range(F):
            pltpu.async_copy(
                w16_hbm.at[idx_v.at[f]], buf_v.at[pl.ds(f * CW // 6, CW // 6)], sem
            )

    def drain(par):
        r_v, idx_v, buf_v, sem = bufs[par]
        for f in range(F):
            pltpu.make_async_copy(
                w16_hbm.at[idx_v.at[f]], buf_v.at[pl.ds(f * CW // 6, CW // 6)], sem
            ).wait()

    def compute(c, par):
        r_v, idx_v, buf_v, sem = bufs[par]
        g = wid * CHUNKS + c
        zero = jnp.zeros((16,), jnp.float32)
        for t in range(T):
            out_v[t] = zero

        def acc_f(f, base0):
            rv = r_v[pl.ds(f * 16, 16)]
            base = base0 + rv
            for t in range(T):
                val = plsc.load_gather(
                    buf_v,
                    [lax.shift_right_logical(base + t, 4),
                     jnp.bitwise_and(base + t, 15)],
                )
                plsc.addupdate(out_v.at[t], val)
            return base0 + B * RPW * 16

        lax.fori_loop(0, F, acc_f, iota * (RPW * 16))
        for m in range(4):
            tvec = iota + 16 * m
            for b in range(B):
                bvec = jnp.full((16,), b, jnp.int32)
                out2_v[b, pl.ds(16 * m, 16)] = plsc.load_gather(out_v, [tvec, bvec])
        pltpu.sync_copy(out2_v, feats_hbm.at[pl.ds(g * B, B)])

    prep(0, 0)
    prep(1, 1)

    def pipe(i, carry):
        drain(0)
        compute(2 * i, 0)
        prep(2 * i + 2, 0)
        drain(1)
        compute(2 * i + 1, 1)
        prep(2 * i + 3, 1)
        return carry

    lax.fori_loop(0, CHUNKS // 2 - 1, pipe, 0)
    drain(0)
    compute(CHUNKS - 2, 0)
    drain(1)
    compute(CHUNKS - 1, 1)


_sc_feats = pl.kernel(
    _sc_body,
    out_type=jax.ShapeDtypeStruct((S * B, T), jnp.float32),
    mesh=plsc.VectorSubcoreMesh(core_axis_name="c", subcore_axis_name="s"),
    scratch_types=[
        pltpu.VMEM((CW,), jnp.int32),
        pltpu.VMEM((CW,), jnp.int32),
        pltpu.VMEM((CW,), jnp.int32),
        pltpu.VMEM((F, B * RPW), jnp.int32),
        pltpu.VMEM((F, B * RPW), jnp.int32),
        pltpu.VMEM((CROWS, 16), jnp.float32),
        pltpu.VMEM((CROWS, 16), jnp.float32),
        pltpu.VMEM((T, B), jnp.float32),
        pltpu.VMEM((B, T), jnp.float32),
        pltpu.SemaphoreType.DMA,
        pltpu.SemaphoreType.DMA,
    ],
    compiler_params=pltpu.CompilerParams(
        needs_layout_passes=False, use_tc_tiling_on_sc=False
    ),
)


def _tc_body(feats_ref, tags_ref, lenc_ref, lenr_ref, trans_ref, out_ref, ef_ref):
    trans = trans_ref[...]
    E = jnp.exp(trans)
    len_col = lenc_ref[...]
    len_row = lenr_ref[...]
    ef_ref[...] = jnp.exp(feats_ref[...])

    K = 8

    def blk(i, carry):
        A, off = carry
        for j in range(K):
            s = i * K + 1 + j
            f_t = ef_ref[jnp.minimum(s, S - 1)]
            An = lax.dot_general(
                A, E, (((1,), (0,)), ((), ())),
                preferred_element_type=jnp.float32,
                precision=lax.Precision.HIGHEST,
            ) * f_t
            A = jnp.where(s < len_col, An, A)
        mx = jnp.max(A, axis=1, keepdims=True)
        return A / mx, off + jnp.log(mx)

    A, off = lax.fori_loop(
        0, S // K, blk, (ef_ref[0], jnp.zeros((B, 1), jnp.float32))
    )
    Z = off + jnp.log(jnp.sum(A, axis=1, keepdims=True))

    tags = tags_ref[...]
    feats = feats_ref[...]
    siota = lax.broadcasted_iota(jnp.int32, (S, B), 0)
    valid = (siota < len_row).astype(jnp.float32)

    tiota = lax.broadcasted_iota(jnp.int32, (S, B, T), 2)
    emit_sel = jnp.sum(
        jnp.where(tiota == tags[:, :, None], feats, 0.0), axis=2
    )

    piota = lax.broadcasted_iota(jnp.int32, (S - 1, B, T), 2)
    ohp = (piota == tags[:-1][:, :, None]).astype(jnp.float32)
    rows = lax.dot_general(
        ohp.reshape((S - 1) * B, T), trans, (((1,), (0,)), ((), ())),
        preferred_element_type=jnp.float32,
        precision=lax.Precision.HIGHEST,
    ).reshape(S - 1, B, T)
    trans_sel = jnp.sum(
        jnp.where(piota == tags[1:][:, :, None], rows, 0.0), axis=2
    )

    gold = emit_sel * valid + jnp.pad(
        trans_sel * valid[1:], ((1, 0), (0, 0))
    )
    ones = jnp.ones((S, 1), jnp.float32)
    gold_col = lax.dot_general(
        gold, ones, (((0,), (0,)), ((), ())),
        preferred_element_type=jnp.float32,
        precision=lax.Precision.HIGHEST,
    )
    out_ref[...] = Z - gold_col


def _tc_crf(feats3, tags, len_col, len_row, trans):
    return pl.pallas_call(
        _tc_body,
        out_shape=jax.ShapeDtypeStruct((B, 1), jnp.float32),
        scratch_shapes=[pltpu.VMEM((S, B, T), jnp.float32)],
    )(feats3, tags, len_col, len_row, trans)


def kernel(input_words, input_tags, input_length, W, trans):
    words_flat = input_words.reshape(-1)
    w16 = W.reshape(FEAT // 16, 16)
    feats = _sc_feats(words_flat, w16)
    feats3 = feats.reshape(S, B, T)
    loss = _tc_crf(
        feats3, input_tags,
        input_length.reshape(B, 1), input_length.reshape(1, B), trans,
    )
    return loss.reshape(B)


# speedup vs baseline: 1.5704x; 1.5704x over previous
import jax
import jax.numpy as jnp
from jax import lax
from jax.experimental import pallas as pl
from jax.experimental.pallas import tpu as pltpu
from jax.experimental.pallas import tpu_sc as plsc

S = 512
B = 16
F = 30
T = 64
FEAT = 1000000

NW = 32
CHUNKS = S // NW
CW = B * F
RPW = 5
CROWS = CW * RPW


def _sc_body(
    words_hbm, w16_hbm, feats_hbm,
    words_v, r0_v, r1_v, idx0_v, idx1_v, buf0_v, buf1_v, out2_v,
    sem0, sem1,
):
    wid = lax.axis_index("s") * 2 + lax.axis_index("c")
    iota = lax.iota(jnp.int32, 16)
    bufs = ((r0_v, idx0_v, buf0_v, sem0), (r1_v, idx1_v, buf1_v, sem1))

    def prep(c, par):
        r_v, idx_v, buf_v, sem = bufs[par]
        g = wid * CHUNKS + c
        pltpu.sync_copy(words_hbm.at[pl.ds(g * CW, CW)], words_v)

        def build_f(f, carry):
            wv = plsc.load_gather(words_v, [iota * F + f])
            q = lax.shift_right_logical(wv, 4)
            r_v[pl.ds(f * 16, 16)] = jnp.bitwise_and(wv, 15)
            fvec = jnp.full((16,), f, jnp.int32)
            for k in range(RPW):
                plsc.store_scatter(idx_v, [fvec, iota * RPW + k], q + k)
            return carry

        lax.fori_loop(0, F, build_f, 0)
        for f in range(F):
            pltpu.async_copy(
                w16_hbm.at[idx_v.at[f]], buf_v.at[pl.ds(f * CW // 6, CW // 6)], sem
            )

    def drain(par):
        r_v, idx_v, buf_v, sem = bufs[par]
        for f in range(F):
            pltpu.make_async_copy(
                w16_hbm.at[idx_v.at[f]], buf_v.at[pl.ds(f * CW // 6, CW // 6)], sem
            ).wait()

    def compute(c, par):
        r_v, idx_v, buf_v, sem = bufs[par]
        g = wid * CHUNKS + c
        for grp in range(4):
            def acc_f(f, carry):
                base0, accs = carry
                rv = r_v[pl.ds(f * 16, 16)]
                base = base0 + rv + (grp * 16)
                out = []
                for tl in range(16):
                    ft = base + tl
                    out.append(accs[tl] + plsc.load_gather(
                        buf_v,
                        [lax.shift_right_logical(ft, 4),
                         jnp.bitwise_and(ft, 15)],
                    ))
                return base0 + B * RPW * 16, tuple(out)

            _, accs = lax.fori_loop(
                0, F, acc_f,
                (iota * (RPW * 16),
                 tuple(jnp.zeros((16,), jnp.float32) for _ in range(16))),
            )
            for tl in range(16):
                tvec = jnp.full((16,), grp * 16 + tl, jnp.int32)
                plsc.store_scatter(out2_v, [iota, tvec], accs[tl])
        pltpu.sync_copy(out2_v, feats_hbm.at[pl.ds(g * B, B)])

    prep(0, 0)
    prep(1, 1)

    def pipe(i, carry):
        drain(0)
        compute(2 * i, 0)
        prep(2 * i + 2, 0)
        drain(1)
        compute(2 * i + 1, 1)
        prep(2 * i + 3, 1)
        return carry

    lax.fori_loop(0, CHUNKS // 2 - 1, pipe, 0)
    drain(0)
    compute(CHUNKS - 2, 0)
    drain(1)
    compute(CHUNKS - 1, 1)


_sc_feats = pl.kernel(
    _sc_body,
    out_type=jax.ShapeDtypeStruct((S * B, T), jnp.float32),
    mesh=plsc.VectorSubcoreMesh(core_axis_name="c", subcore_axis_name="s"),
    scratch_types=[
        pltpu.VMEM((CW,), jnp.int32),
        pltpu.VMEM((CW,), jnp.int32),
        pltpu.VMEM((CW,), jnp.int32),
        pltpu.VMEM((F, B * RPW), jnp.int32),
        pltpu.VMEM((F, B * RPW), jnp.int32),
        pltpu.VMEM((CROWS, 16), jnp.float32),
        pltpu.VMEM((CROWS, 16), jnp.float32),
        pltpu.VMEM((B, T), jnp.float32),
        pltpu.SemaphoreType.DMA,
        pltpu.SemaphoreType.DMA,
    ],
    compiler_params=pltpu.CompilerParams(
        needs_layout_passes=False, use_tc_tiling_on_sc=False
    ),
)


def _tc_body(feats_ref, tags_ref, lenc_ref, lenr_ref, trans_ref, out_ref, ef_ref):
    trans = trans_ref[...]
    E = jnp.exp(trans)
    len_col = lenc_ref[...]
    len_row = lenr_ref[...]
    ef_ref[...] = jnp.exp(feats_ref[...])

    K = 8

    def blk(i, carry):
        A, off = carry
        for j in range(K):
            s = i * K + 1 + j
            f_t = ef_ref[jnp.minimum(s, S - 1)]
            An = lax.dot_general(
                A, E, (((1,), (0,)), ((), ())),
                preferred_element_type=jnp.float32,
                precision=lax.Precision.HIGHEST,
            ) * f_t
            A = jnp.where(s < len_col, An, A)
        mx = jnp.max(A, axis=1, keepdims=True)
        return A / mx, off + jnp.log(mx)

    A, off = lax.fori_loop(
        0, S // K, blk, (ef_ref[0], jnp.zeros((B, 1), jnp.float32))
    )
    Z = off + jnp.log(jnp.sum(A, axis=1, keepdims=True))

    tags = tags_ref[...]
    feats = feats_ref[...]
    siota = lax.broadcasted_iota(jnp.int32, (S, B), 0)
    valid = (siota < len_row).astype(jnp.float32)

    tiota = lax.broadcasted_iota(jnp.int32, (S, B, T), 2)
    emit_sel = jnp.sum(
        jnp.where(tiota == tags[:, :, None], feats, 0.0), axis=2
    )

    piota = lax.broadcasted_iota(jnp.int32, (S - 1, B, T), 2)
    ohp = (piota == tags[:-1][:, :, None]).astype(jnp.float32)
    rows = lax.dot_general(
        ohp.reshape((S - 1) * B, T), trans, (((1,), (0,)), ((), ())),
        preferred_element_type=jnp.float32,
        precision=lax.Precision.HIGHEST,
    ).reshape(S - 1, B, T)
    trans_sel = jnp.sum(
        jnp.where(piota == tags[1:][:, :, None], rows, 0.0), axis=2
    )

    gold = emit_sel * valid + jnp.pad(
        trans_sel * valid[1:], ((1, 0), (0, 0))
    )
    ones = jnp.ones((S, 1), jnp.float32)
    gold_col = lax.dot_general(
        gold, ones, (((0,), (0,)), ((), ())),
        preferred_element_type=jnp.float32,
        precision=lax.Precision.HIGHEST,
    )
    out_ref[...] = Z - gold_col


def _tc_crf(feats3, tags, len_col, len_row, trans):
    return pl.pallas_call(
        _tc_body,
        out_shape=jax.ShapeDtypeStruct((B, 1), jnp.float32),
        scratch_shapes=[pltpu.VMEM((S, B, T), jnp.float32)],
    )(feats3, tags, len_col, len_row, trans)


def kernel(input_words, input_tags, input_length, W, trans):
    words_flat = input_words.reshape(-1)
    w16 = W.reshape(FEAT // 16, 16)
    feats = _sc_feats(words_flat, w16)
    feats3 = feats.reshape(S, B, T)
    loss = _tc_crf(
        feats3, input_tags,
        input_length.reshape(B, 1), input_length.reshape(1, B), trans,
    )
    return loss.reshape(B)
